# baseline (speedup 1.0000x reference)
import functools

import jax
import jax.numpy as jnp
from jax import lax
from jax.experimental import pallas as pl
from jax.experimental.pallas import tpu as pltpu
from jax.experimental.pallas import tpu_sc as plsc

N = 10000
E = 160000
D = 256
H = 128
DE = 16

NW = 32
R = E // 128
ROWS_PER = 40
R_PAD = NW * ROWS_PER
E_PAD = R_PAD * 128
NHALF = N // 2
NQ = N // 4
SROWS = R_PAD // 16
ACC_ROWS = 2560

F32 = jnp.float32
BF16 = jnp.bfloat16


def _elu(v):
    return jnp.where(v > 0, v, jnp.exp(jnp.minimum(v, 0.0)) - 1.0)


def _lrelu(v):
    return jnp.where(v > 0, v, 0.01 * v)


def _dot(a, b):
    return jax.lax.dot_general(a, b, (((1,), (0,)), ((), ())),
                               preferred_element_type=F32)


def _transfer(x, w1, w2):
    out = _elu(_dot(x, w1.T))
    mu = jnp.mean(out, axis=-1, keepdims=True)
    var = jnp.var(out, axis=-1, keepdims=True)
    out = (out - mu) / jnp.sqrt(var + 1e-5)
    return _elu(_dot(out, w2.T) + x)


def _prep0_body(x, w1, w2, wi, bi, wj, bj, wk, bk, wr,
                t0, xi, xj, xk, xr):
    t = _transfer(x[...], w1[...], w2[...])
    t0[...] = t
    xi[...] = _dot(t, wi[...].T) + bi[...]
    xj[...] = _dot(t, wj[...].T) + bj[...]
    xk[...] = _dot(t, wk[...].T) + bk[...]
    xr[...] = _dot(t, wr[...].T)


def _msg_body(g0, g1, g2, xrg, ha, we, be, wl, att, m_out, ex_out):
    e = _dot(ha[...], we[...].T) + be[...]
    m = _elu(g0[...] + g1[...] + g2[...] + e)
    m_out[...] = m.astype(BF16)
    q = _lrelu(_dot(m, wl[...].T) + xrg[...])
    s = jnp.sum(q * att[...], axis=1, keepdims=True)
    ex_out[...] = jnp.exp(_lrelu(s))


def _reduce_body(sp, gp, ssum, shw):
    ssum[...] = jnp.sum(sp[...], axis=0)
    shw[...] = jnp.sum(gp[...], axis=0)


def _scale_body(m, al, out):
    out[...] = m[...].astype(F32) * al[...]


def _update1_body(xp, acc, w2, cb, wi, bi, wj, bj, wk, bk, wr,
                  x1, xi, xj, xk, xr):
    a = acc[...]
    t = _elu(xp[...] + _dot(a, w2[...].T) + cb[...])
    x1[...] = t
    xi[...] = _dot(t, wi[...].T) + bi[...]
    xj[...] = _dot(t, wj[...].T) + bj[...]
    xk[...] = _dot(t, wk[...].T) + bk[...]
    xr[...] = _dot(t, wr[...].T)


def _update2_body(xp, acc, w2, cb, tw1, tw2, ew, out):
    a = acc[...]
    t = _elu(xp[...] + _dot(a, w2[...].T) + cb[...])
    t = _transfer(t, tw1[...], tw2[...])
    out[...] = _dot(t, ew[...].T)


_BN = 1000
_BE = 6400


def _full(shape):
    return pl.BlockSpec(shape, lambda i: tuple(0 for _ in shape))


def _tc_prep0(x, p):
    grid = (N // _BN,)
    return pl.pallas_call(
        _prep0_body,
        grid=grid,
        in_specs=[pl.BlockSpec((_BN, D), lambda i: (i, 0)),
                  _full((D, D)), _full((D, D)),
                  _full((H, D)), _full((1, H)),
                  _full((H, D)), _full((1, H)),
                  _full((H, D)), _full((1, H)),
                  _full((H, D))],
        out_specs=[pl.BlockSpec((_BN, D), lambda i: (i, 0))] +
                  [pl.BlockSpec((_BN, H), lambda i: (i, 0))] * 4,
        out_shape=[jax.ShapeDtypeStruct((N, D), F32)] +
                  [jax.ShapeDtypeStruct((N, H), F32)] * 4,
    )(x, p["trans0_W1"], p["trans0_W2"],
      p["outp_Wi"], p["outp_bi"].reshape(1, H),
      p["outp_Wj"], p["outp_bj"].reshape(1, H),
      p["outp_Wk"], p["outp_bk"].reshape(1, H),
      p["outp_Wr"])


def _tc_msg(g0, g1, g2, xrg, ha, p, pre):
    grid = (E // _BE,)
    return pl.pallas_call(
        _msg_body,
        grid=grid,
        in_specs=[pl.BlockSpec((_BE, H), lambda i: (i, 0))] * 4 +
                 [pl.BlockSpec((_BE, DE), lambda i: (i, 0)),
                  _full((H, DE)), _full((1, H)), _full((H, H)),
                  _full((1, H))],
        out_specs=[pl.BlockSpec((_BE, H), lambda i: (i, 0)),
                   pl.BlockSpec((_BE, 1), lambda i: (i, 0))],
        out_shape=[jax.ShapeDtypeStruct((E_PAD, H), BF16),
                   jax.ShapeDtypeStruct((E_PAD, 1), F32)],
    )(g0, g1, g2, xrg, ha,
      p[pre + "_We"], p[pre + "_be"].reshape(1, H),
      p[pre + "_Wl"], p[pre + "_att"].reshape(1, H))


def _tc_reduce(sp, gp):
    return pl.pallas_call(
        _reduce_body,
        out_shape=[jax.ShapeDtypeStruct((N,), F32)] * 2,
    )(sp, gp)


def _tc_scale(m, al):
    grid = (E // _BE,)
    return pl.pallas_call(
        _scale_body,
        grid=grid,
        in_specs=[pl.BlockSpec((_BE, H), lambda i: (i, 0)),
                  pl.BlockSpec((_BE, 1), lambda i: (i, 0))],
        out_specs=pl.BlockSpec((_BE, H), lambda i: (i, 0)),
        out_shape=jax.ShapeDtypeStruct((E_PAD, H), F32),
    )(m, al)


def _tc_update1(xp, acc, p):
    grid = (N // _BN,)
    return pl.pallas_call(
        _update1_body,
        grid=grid,
        in_specs=[pl.BlockSpec((_BN, D), lambda i: (i, 0)),
                  pl.BlockSpec((_BN, H), lambda i: (i, 0)),
                  _full((D, H)), _full((1, D)),
                  _full((H, D)), _full((1, H)),
                  _full((H, D)), _full((1, H)),
                  _full((H, D)), _full((1, H)),
                  _full((H, D))],
        out_specs=[pl.BlockSpec((_BN, D), lambda i: (i, 0))] +
                  [pl.BlockSpec((_BN, H), lambda i: (i, 0))] * 4,
        out_shape=[jax.ShapeDtypeStruct((N, D), F32)] +
                  [jax.ShapeDtypeStruct((N, H), F32)] * 4,
    )(xp, acc, p["outp_W2"], p["outp_cbias"].reshape(1, D),
      p["inp_Wi"], p["inp_bi"].reshape(1, H),
      p["inp_Wj"], p["inp_bj"].reshape(1, H),
      p["inp_Wk"], p["inp_bk"].reshape(1, H),
      p["inp_Wr"])


def _tc_update2(xp, acc, p):
    grid = (N // _BN,)
    return pl.pallas_call(
        _update2_body,
        grid=grid,
        in_specs=[pl.BlockSpec((_BN, D), lambda i: (i, 0)),
                  pl.BlockSpec((_BN, H), lambda i: (i, 0)),
                  _full((D, H)), _full((1, D)),
                  _full((D, D)), _full((D, D)), _full((D, D))],
        out_specs=pl.BlockSpec((_BN, D), lambda i: (i, 0)),
        out_shape=jax.ShapeDtypeStruct((N, D), F32),
    )(xp, acc, p["inp_W2"], p["inp_cbias"].reshape(1, D),
      p["trans1_W1"], p["trans1_W2"], p["exit_W"])


_MESH = plsc.VectorSubcoreMesh(core_axis_name="c", subcore_axis_name="s")
_SC_PARAMS = pltpu.CompilerParams(needs_layout_passes=False)


def _wid():
    return lax.axis_index("s") * 2 + lax.axis_index("c")


def _stage_rows(src2d, dst, w):
    pltpu.sync_copy(src2d.at[pl.ds(w * ROWS_PER, ROWS_PER)], dst)


def _nrows(w):
    return jnp.clip(R - w * ROWS_PER, 0, ROWS_PER).astype(jnp.int32)


def _gather_body(uv, i0, i1, i2, ti, tj, tk, tr,
                 g0, g1, g2, xrout,
                 i0b, i1b, i2b,
                 ba0, ba1, ba2, ba3, bb0, bb1, bb2, bb3,
                 semga, semgb, semwa, semwb):
    w = _wid()
    _stage_rows(i0, i0b, w)
    _stage_rows(i1, i1b, w)
    _stage_rows(i2, i2b, w)
    iuvb = (i0b, i1b, i2b)[uv]
    ncc = _nrows(w) * 2
    bufs = ((ba0, ba1, ba2, ba3), (bb0, bb1, bb2, bb3))
    gsems = (semga, semgb)
    wsems = (semwa, semwb)
    outs = (g0, g1, g2, xrout)

    def srcs(cc):
        j = cc // 2
        sl = pl.ds((cc % 2) * 64, 64)
        return (ti.at[i0b.at[j, sl]], tj.at[i1b.at[j, sl]],
                tk.at[i2b.at[j, sl]], tr.at[iuvb.at[j, sl]])

    def pair(j2, carry):
        js = (2 * j2, 2 * j2 + 1)
        for b in range(2):
            @pl.when(js[b] < ncc)
            def _(b=b):
                for s_, b_ in zip(srcs(js[b]), bufs[b]):
                    pltpu.async_copy(s_, b_, gsems[b])
        for b in range(2):
            @pl.when(js[b] < ncc)
            def _(b=b):
                for s_, b_ in zip(srcs(js[b]), bufs[b]):
                    pltpu.make_async_copy(s_, b_, gsems[b]).wait()
                base = w * ROWS_PER * 128 + js[b] * 64
                for o_, b_ in zip(outs, bufs[b]):
                    pltpu.async_copy(b_, o_.at[pl.ds(base, 64)], wsems[b])
        for b in range(2):
            @pl.when(js[b] < ncc)
            def _(b=b):
                base = w * ROWS_PER * 128 + js[b] * 64
                for o_, b_ in zip(outs, bufs[b]):
                    pltpu.make_async_copy(
                        b_, o_.at[pl.ds(base, 64)], wsems[b]).wait()
        return carry

    lax.fori_loop(0, ROWS_PER, pair, 0)


def _sc_gather(uv, i0, i1, i2, ti, tj, tk, tr):
    fn = pl.kernel(
        functools.partial(_gather_body, uv),
        out_type=[jax.ShapeDtypeStruct((E_PAD, H), F32)] * 4,
        mesh=_MESH,
        scratch_types=[pltpu.VMEM((ROWS_PER, 128), jnp.int32)] * 3 +
                      [pltpu.VMEM((64, H), F32)] * 8 +
                      [pltpu.SemaphoreType.DMA] * 4,
        compiler_params=_SC_PARAMS,
    )
    return fn(i0, i1, i2, ti, tj, tk, tr)


def _sums_body(ex2, hw2, ix2, out, exb, hwb, ixb, s1, s2):
    w = _wid()
    zero16 = jnp.zeros((16,), F32)

    def z(i, carry):
        s1[pl.ds(i * 16, 16)] = zero16
        s2[pl.ds(i * 16, 16)] = zero16
        return carry

    lax.fori_loop(0, N // 16, z, 0)
    _stage_rows(ex2, exb, w)
    _stage_rows(hw2, hwb, w)
    _stage_rows(ix2, ixb, w)

    def row(j, carry):
        for k in range(8):
            sl = pl.ds(k * 16, 16)
            i16 = ixb[j, sl]
            e16 = exb[j, sl]
            h16 = hwb[j, sl]
            plsc.addupdate_scatter(s1, [i16], e16)
            plsc.addupdate_scatter(s2, [i16], e16 * h16)
        return carry

    lax.fori_loop(0, _nrows(w), row, 0)
    pltpu.sync_copy(s1, out.at[pl.ds(w * N, N)])
    pltpu.sync_copy(s2, out.at[pl.ds((NW + w) * N, N)])


def _sc_sums(ex2, hw2, ix2):
    fn = pl.kernel(
        _sums_body,
        out_type=jax.ShapeDtypeStruct((2 * NW * N,), F32),
        mesh=_MESH,
        scratch_types=[pltpu.VMEM((ROWS_PER, 128), F32),
                       pltpu.VMEM((ROWS_PER, 128), F32),
                       pltpu.VMEM((ROWS_PER, 128), jnp.int32),
                       pltpu.VMEM((N,), F32),
                       pltpu.VMEM((N,), F32)],
        compiler_params=_SC_PARAMS,
    )
    return fn(ex2, hw2, ix2)


def _alpha_body(ex2, hw2, ix2, ssum, shw, alpha,
                exb, hwb, ixb, sN, gN, abuf):
    w = _wid()
    pltpu.sync_copy(ssum, sN)
    pltpu.sync_copy(shw, gN)
    _stage_rows(ex2, exb, w)
    _stage_rows(hw2, hwb, w)
    _stage_rows(ix2, ixb, w)

    def row(j, carry):
        for k in range(8):
            sl = pl.ds(k * 16, 16)
            i16 = ixb[j, sl]
            e16 = exb[j, sl]
            h16 = hwb[j, sl]
            sg = plsc.load_gather(sN, [i16])
            gg = plsc.load_gather(gN, [i16])
            abuf[sl] = e16 * h16 / (gg + 1e-5 * sg + 1e-21)
        base = (w * ROWS_PER + j) * 128
        pltpu.sync_copy(abuf, alpha.at[pl.ds(base, 128)])
        return carry

    lax.fori_loop(0, _nrows(w), row, 0)


def _sc_alpha(ex2, hw2, ix2, ssum, shw):
    fn = pl.kernel(
        _alpha_body,
        out_type=jax.ShapeDtypeStruct((E_PAD,), F32),
        mesh=_MESH,
        scratch_types=[pltpu.VMEM((ROWS_PER, 128), F32),
                       pltpu.VMEM((ROWS_PER, 128), F32),
                       pltpu.VMEM((ROWS_PER, 128), jnp.int32),
                       pltpu.VMEM((N,), F32),
                       pltpu.VMEM((N,), F32),
                       pltpu.VMEM((128,), F32)],
        compiler_params=_SC_PARAMS,
    )
    return fn(ex2, hw2, ix2, ssum, shw)


def _scatter_body(m, ix2, acc, ixb, ixt0, ixt1, mb0, mb1,
                  semr0, semr1, sems0, sems1, spacc):
    c = lax.axis_index("c")
    s = lax.axis_index("s")

    pltpu.sync_copy(ix2.at[pl.ds(s * SROWS, SROWS)], ixb)
    nrows = jnp.clip(R - s * SROWS, 0, SROWS).astype(jnp.int32)
    mbs = (mb0, mb1)
    ixts = (ixt0, ixt1)
    rsems = (semr0, semr1)
    ssems = (sems0, sems1)
    zero16 = jnp.zeros((16,), F32)

    for p in range(2):
        lo = c * NHALF + p * NQ

        def zm(i, carry):
            for k in range(8):
                mb0[i, pl.ds(k * 16, 16)] = zero16
            return carry

        lax.fori_loop(0, 128, zm, 0)
        pltpu.sync_copy(mb0, spacc.at[pl.ds(s * 160, 128)])
        pltpu.sync_copy(mb0.at[pl.ds(0, 32)],
                        spacc.at[pl.ds(s * 160 + 128, 32)])
        plsc.subcore_barrier()

        def pair(j2, carry):
            js = (2 * j2, 2 * j2 + 1)
            for b in range(2):
                @pl.when(js[b] < nrows)
                def _(b=b):
                    base = (s * SROWS + js[b]) * 128
                    pltpu.async_copy(m.at[pl.ds(base, 128)], mbs[b],
                                     rsems[b])
            for b in range(2):
                @pl.when(js[b] < nrows)
                def _(b=b):
                    for k in range(8):
                        sl = pl.ds(k * 16, 16)
                        i16 = ixb[js[b], sl]
                        iloc = i16 - lo
                        ok = (iloc >= 0) & (iloc < NQ)
                        ixts[b][sl] = jnp.where(ok, iloc, NQ + 32)
                    base = (s * SROWS + js[b]) * 128
                    pltpu.make_async_copy(
                        m.at[pl.ds(base, 128)], mbs[b], rsems[b]).wait()
                    pltpu.async_copy(mbs[b], spacc.at[ixts[b]], ssems[b],
                                     add=True)
            for b in range(2):
                @pl.when(js[b] < nrows)
                def _(b=b):
                    pltpu.make_async_copy(mbs[b], spacc.at[ixts[b]],
                                          ssems[b]).wait()
            return carry

        lax.fori_loop(0, SROWS // 2, pair, 0)
        plsc.subcore_barrier()
        q = 2 * c + p
        start = s * 160
        pltpu.sync_copy(spacc.at[pl.ds(start, 128)], mb0)
        pltpu.sync_copy(mb0, acc.at[q, pl.ds(start, 128)])
        pltpu.sync_copy(spacc.at[pl.ds(start + 128, 32)],
                        mb0.at[pl.ds(0, 32)])
        pltpu.sync_copy(mb0.at[pl.ds(0, 32)],
                        acc.at[q, pl.ds(start + 128, 32)])
        plsc.subcore_barrier()


def _sc_scatter(m, ix2):
    fn = pl.kernel(
        _scatter_body,
        out_type=jax.ShapeDtypeStruct((4, ACC_ROWS, H), F32),
        mesh=_MESH,
        scratch_types=[pltpu.VMEM((SROWS, 128), jnp.int32),
                       pltpu.VMEM((128,), jnp.int32),
                       pltpu.VMEM((128,), jnp.int32),
                       pltpu.VMEM((128, H), F32),
                       pltpu.VMEM((128, H), F32),
                       pltpu.SemaphoreType.DMA,
                       pltpu.SemaphoreType.DMA,
                       pltpu.SemaphoreType.DMA,
                       pltpu.SemaphoreType.DMA,
                       pltpu.VMEM_SHARED((ACC_ROWS, H), F32)],
        compiler_params=_SC_PARAMS,
    )
    return fn(m, ix2)


def _tri_stage(x_tables, hi2d, ha, hw2, uv, p, pre):
    t, ti, tj, tk, tr = x_tables
    i0, i1, i2 = hi2d
    g0, g1, g2, xrg = _sc_gather(uv, i0, i1, i2, ti, tj, tk, tr)
    m, ex = _tc_msg(g0, g1, g2, xrg, ha, p, pre)
    ex2 = ex.reshape(R_PAD, 128)
    ix2 = (i0, i1, i2)[uv]
    parts = _sc_sums(ex2, hw2, ix2)
    sp = parts[:NW * N].reshape(NW, N)
    gp = parts[NW * N:].reshape(NW, N)
    ssum, shw = _tc_reduce(sp, gp)
    alpha = _sc_alpha(ex2, hw2, ix2, ssum, shw)
    msc = _tc_scale(m, alpha.reshape(E_PAD, 1))
    accq = _sc_scatter(msc, ix2)
    acc = accq[:, :NQ, :].reshape(N, H)
    return t, acc


def kernel(x, hyperedge_index, hyperedge_attr, hyperedge_weight, params):
    p = params
    hi = hyperedge_index.astype(jnp.int32)
    pad2d = lambda a: jnp.pad(a.reshape(R, 128), ((0, R_PAD - R), (0, 0)))
    hi2d = tuple(pad2d(hi[v]) for v in range(3))
    hw2 = pad2d(hyperedge_weight)

    t0, xi, xj, xk, xr = _tc_prep0(x, p)
    t0, acc1 = _tri_stage((t0, xi, xj, xk, xr), hi2d, hyperedge_attr,
                          hw2, 2, p, "outp")
    x1, yi, yj, yk, yr = _tc_update1(t0, acc1, p)
    x1, acc2 = _tri_stage((x1, yi, yj, yk, yr), hi2d, hyperedge_attr,
                          hw2, 0, p, "inp")
    return _tc_update2(x1, acc2, p)


# speedup vs baseline: 1.2332x; 1.2332x over previous
import functools

import jax
import jax.numpy as jnp
from jax import lax
from jax.experimental import pallas as pl
from jax.experimental.pallas import tpu as pltpu
from jax.experimental.pallas import tpu_sc as plsc

N = 10000
E = 160000
D = 256
H = 128
DE = 16

NW = 32
R = E // 128
ROWS_PER = 40
R_PAD = NW * ROWS_PER
E_PAD = R_PAD * 128
NHALF = N // 2
SROWS = R_PAD // 16
ACC_ROWS = 5248

F32 = jnp.float32
BF16 = jnp.bfloat16


def _elu(v):
    return jnp.where(v > 0, v, jnp.exp(jnp.minimum(v, 0.0)) - 1.0)


def _lrelu(v):
    return jnp.where(v > 0, v, 0.01 * v)


def _dot(a, b):
    return jax.lax.dot_general(a, b, (((1,), (0,)), ((), ())),
                               preferred_element_type=F32)


def _transfer(x, w1, w2):
    out = _elu(_dot(x, w1.T))
    mu = jnp.mean(out, axis=-1, keepdims=True)
    var = jnp.var(out, axis=-1, keepdims=True)
    out = (out - mu) / jnp.sqrt(var + 1e-5)
    return _elu(_dot(out, w2.T) + x)


def _prep0_body(x, w1, w2, wi, bi, wj, bj, wk, bk, wr,
                t0, xi, xj, xk, xr):
    t = _transfer(x[...], w1[...], w2[...])
    t0[...] = t
    xi[...] = _dot(t, wi[...].T) + bi[...]
    xj[...] = _dot(t, wj[...].T) + bj[...]
    xk[...] = _dot(t, wk[...].T) + bk[...]
    xr[...] = _dot(t, wr[...].T)


def _msg_body(g0, g1, g2, xrg, ha, we, be, wl, att, m_out, ex_out):
    e = _dot(ha[...], we[...].T) + be[...]
    m = _elu(g0[...] + g1[...] + g2[...] + e)
    m_out[...] = m.astype(BF16)
    q = _lrelu(_dot(m, wl[...].T) + xrg[...])
    s = jnp.sum(q * att[...], axis=1, keepdims=True)
    ex_out[...] = jnp.exp(_lrelu(s))


def _reduce_body(sp, gp, ssum, shw):
    ssum[...] = jnp.sum(sp[...], axis=0)
    shw[...] = jnp.sum(gp[...], axis=0)


def _scale_body(m, al, out):
    out[...] = m[...].astype(F32) * al[...]


def _update1_body(xp, acc, w2, cb, wi, bi, wj, bj, wk, bk, wr,
                  x1, xi, xj, xk, xr):
    a = acc[0]
    t = _elu(xp[...] + _dot(a, w2[...].T) + cb[...])
    x1[...] = t
    xi[...] = _dot(t, wi[...].T) + bi[...]
    xj[...] = _dot(t, wj[...].T) + bj[...]
    xk[...] = _dot(t, wk[...].T) + bk[...]
    xr[...] = _dot(t, wr[...].T)


def _update2_body(xp, acc, w2, cb, tw1, tw2, ew, out):
    a = acc[0]
    t = _elu(xp[...] + _dot(a, w2[...].T) + cb[...])
    t = _transfer(t, tw1[...], tw2[...])
    out[...] = _dot(t, ew[...].T)


_BN = 1000
_BE = 6400


def _full(shape):
    return pl.BlockSpec(shape, lambda i: tuple(0 for _ in shape))


def _tc_prep0(x, p):
    grid = (N // _BN,)
    return pl.pallas_call(
        _prep0_body,
        grid=grid,
        in_specs=[pl.BlockSpec((_BN, D), lambda i: (i, 0)),
                  _full((D, D)), _full((D, D)),
                  _full((H, D)), _full((1, H)),
                  _full((H, D)), _full((1, H)),
                  _full((H, D)), _full((1, H)),
                  _full((H, D))],
        out_specs=[pl.BlockSpec((_BN, D), lambda i: (i, 0))] +
                  [pl.BlockSpec((_BN, H), lambda i: (i, 0))] * 4,
        out_shape=[jax.ShapeDtypeStruct((N, D), F32)] +
                  [jax.ShapeDtypeStruct((N, H), F32)] * 4,
    )(x, p["trans0_W1"], p["trans0_W2"],
      p["outp_Wi"], p["outp_bi"].reshape(1, H),
      p["outp_Wj"], p["outp_bj"].reshape(1, H),
      p["outp_Wk"], p["outp_bk"].reshape(1, H),
      p["outp_Wr"])


def _tc_msg(g0, g1, g2, xrg, ha, p, pre):
    grid = (E // _BE,)
    return pl.pallas_call(
        _msg_body,
        grid=grid,
        in_specs=[pl.BlockSpec((_BE, H), lambda i: (i, 0))] * 4 +
                 [pl.BlockSpec((_BE, DE), lambda i: (i, 0)),
                  _full((H, DE)), _full((1, H)), _full((H, H)),
                  _full((1, H))],
        out_specs=[pl.BlockSpec((_BE, H), lambda i: (i, 0)),
                   pl.BlockSpec((_BE, 1), lambda i: (i, 0))],
        out_shape=[jax.ShapeDtypeStruct((E_PAD, H), BF16),
                   jax.ShapeDtypeStruct((E_PAD, 1), F32)],
    )(g0, g1, g2, xrg, ha,
      p[pre + "_We"], p[pre + "_be"].reshape(1, H),
      p[pre + "_Wl"], p[pre + "_att"].reshape(1, H))


def _tc_reduce(sp, gp):
    return pl.pallas_call(
        _reduce_body,
        out_shape=[jax.ShapeDtypeStruct((N,), F32)] * 2,
    )(sp, gp)


def _tc_scale(m, al):
    grid = (E // _BE,)
    return pl.pallas_call(
        _scale_body,
        grid=grid,
        in_specs=[pl.BlockSpec((_BE, H), lambda i: (i, 0)),
                  pl.BlockSpec((_BE, 1), lambda i: (i, 0))],
        out_specs=pl.BlockSpec((_BE, H), lambda i: (i, 0)),
        out_shape=jax.ShapeDtypeStruct((E_PAD, H), F32),
    )(m, al)


def _tc_update1(xp, acc, p):
    grid = (N // _BN,)
    return pl.pallas_call(
        _update1_body,
        grid=grid,
        in_specs=[pl.BlockSpec((_BN, D), lambda i: (i, 0)),
                  pl.BlockSpec((1, _BN, H), lambda i: (i // 5, i % 5, 0)),
                  _full((D, H)), _full((1, D)),
                  _full((H, D)), _full((1, H)),
                  _full((H, D)), _full((1, H)),
                  _full((H, D)), _full((1, H)),
                  _full((H, D))],
        out_specs=[pl.BlockSpec((_BN, D), lambda i: (i, 0))] +
                  [pl.BlockSpec((_BN, H), lambda i: (i, 0))] * 4,
        out_shape=[jax.ShapeDtypeStruct((N, D), F32)] +
                  [jax.ShapeDtypeStruct((N, H), F32)] * 4,
    )(xp, acc, p["outp_W2"], p["outp_cbias"].reshape(1, D),
      p["inp_Wi"], p["inp_bi"].reshape(1, H),
      p["inp_Wj"], p["inp_bj"].reshape(1, H),
      p["inp_Wk"], p["inp_bk"].reshape(1, H),
      p["inp_Wr"])


def _tc_update2(xp, acc, p):
    grid = (N // _BN,)
    return pl.pallas_call(
        _update2_body,
        grid=grid,
        in_specs=[pl.BlockSpec((_BN, D), lambda i: (i, 0)),
                  pl.BlockSpec((1, _BN, H), lambda i: (i // 5, i % 5, 0)),
                  _full((D, H)), _full((1, D)),
                  _full((D, D)), _full((D, D)), _full((D, D))],
        out_specs=pl.BlockSpec((_BN, D), lambda i: (i, 0)),
        out_shape=jax.ShapeDtypeStruct((N, D), F32),
    )(xp, acc, p["inp_W2"], p["inp_cbias"].reshape(1, D),
      p["trans1_W1"], p["trans1_W2"], p["exit_W"])


_MESH = plsc.VectorSubcoreMesh(core_axis_name="c", subcore_axis_name="s")
_SC_PARAMS = pltpu.CompilerParams(needs_layout_passes=False)


def _wid():
    return lax.axis_index("s") * 2 + lax.axis_index("c")


def _stage_rows(src2d, dst, w):
    pltpu.sync_copy(src2d.at[pl.ds(w * ROWS_PER, ROWS_PER)], dst)


def _nrows(w):
    return jnp.clip(R - w * ROWS_PER, 0, ROWS_PER).astype(jnp.int32)


def _gather_body(uv, i0, i1, i2, ti, tj, tk, tr,
                 g0, g1, g2, xrout,
                 i0b, i1b, i2b,
                 ba0, ba1, ba2, ba3, bb0, bb1, bb2, bb3,
                 semga, semgb, semwa, semwb):
    w = _wid()
    _stage_rows(i0, i0b, w)
    _stage_rows(i1, i1b, w)
    _stage_rows(i2, i2b, w)
    iuvb = (i0b, i1b, i2b)[uv]
    ncc = _nrows(w) * 2
    bufs = ((ba0, ba1, ba2, ba3), (bb0, bb1, bb2, bb3))
    gsems = (semga, semgb)
    wsems = (semwa, semwb)
    outs = (g0, g1, g2, xrout)

    def srcs(cc):
        j = cc // 2
        sl = pl.ds((cc % 2) * 64, 64)
        return (ti.at[i0b.at[j, sl]], tj.at[i1b.at[j, sl]],
                tk.at[i2b.at[j, sl]], tr.at[iuvb.at[j, sl]])

    def pair(j2, carry):
        js = (2 * j2, 2 * j2 + 1)
        for b in range(2):
            @pl.when(js[b] < ncc)
            def _(b=b):
                for s_, b_ in zip(srcs(js[b]), bufs[b]):
                    pltpu.async_copy(s_, b_, gsems[b])
        for b in range(2):
            @pl.when(js[b] < ncc)
            def _(b=b):
                for s_, b_ in zip(srcs(js[b]), bufs[b]):
                    pltpu.make_async_copy(s_, b_, gsems[b]).wait()
                base = w * ROWS_PER * 128 + js[b] * 64
                for o_, b_ in zip(outs, bufs[b]):
                    pltpu.async_copy(b_, o_.at[pl.ds(base, 64)], wsems[b])
        for b in range(2):
            @pl.when(js[b] < ncc)
            def _(b=b):
                base = w * ROWS_PER * 128 + js[b] * 64
                for o_, b_ in zip(outs, bufs[b]):
                    pltpu.make_async_copy(
                        b_, o_.at[pl.ds(base, 64)], wsems[b]).wait()
        return carry

    lax.fori_loop(0, ROWS_PER, pair, 0)


def _sc_gather(uv, i0, i1, i2, ti, tj, tk, tr):
    fn = pl.kernel(
        functools.partial(_gather_body, uv),
        out_type=[jax.ShapeDtypeStruct((E_PAD, H), F32)] * 4,
        mesh=_MESH,
        scratch_types=[pltpu.VMEM((ROWS_PER, 128), jnp.int32)] * 3 +
                      [pltpu.VMEM((64, H), F32)] * 8 +
                      [pltpu.SemaphoreType.DMA] * 4,
        compiler_params=_SC_PARAMS,
    )
    return fn(i0, i1, i2, ti, tj, tk, tr)


def _sums_body(ex2, hw2, ix2, out, exb, hwb, ixb, s1, s2):
    w = _wid()
    zero16 = jnp.zeros((16,), F32)

    def z(i, carry):
        s1[pl.ds(i * 16, 16)] = zero16
        s2[pl.ds(i * 16, 16)] = zero16
        return carry

    lax.fori_loop(0, N // 16, z, 0)
    _stage_rows(ex2, exb, w)
    _stage_rows(hw2, hwb, w)
    _stage_rows(ix2, ixb, w)

    def row(j, carry):
        for k in range(8):
            sl = pl.ds(k * 16, 16)
            i16 = ixb[j, sl]
            e16 = exb[j, sl]
            h16 = hwb[j, sl]
            plsc.addupdate_scatter(s1, [i16], e16)
            plsc.addupdate_scatter(s2, [i16], e16 * h16)
        return carry

    lax.fori_loop(0, _nrows(w), row, 0)
    pltpu.sync_copy(s1, out.at[pl.ds(w * N, N)])
    pltpu.sync_copy(s2, out.at[pl.ds((NW + w) * N, N)])


def _sc_sums(ex2, hw2, ix2):
    fn = pl.kernel(
        _sums_body,
        out_type=jax.ShapeDtypeStruct((2 * NW * N,), F32),
        mesh=_MESH,
        scratch_types=[pltpu.VMEM((ROWS_PER, 128), F32),
                       pltpu.VMEM((ROWS_PER, 128), F32),
                       pltpu.VMEM((ROWS_PER, 128), jnp.int32),
                       pltpu.VMEM((N,), F32),
                       pltpu.VMEM((N,), F32)],
        compiler_params=_SC_PARAMS,
    )
    return fn(ex2, hw2, ix2)


def _alpha_body(ex2, hw2, ix2, ssum, shw, alpha,
                exb, hwb, ixb, sN, gN, abuf):
    w = _wid()
    pltpu.sync_copy(ssum, sN)
    pltpu.sync_copy(shw, gN)
    _stage_rows(ex2, exb, w)
    _stage_rows(hw2, hwb, w)
    _stage_rows(ix2, ixb, w)

    def row(j, carry):
        for k in range(8):
            sl = pl.ds(k * 16, 16)
            i16 = ixb[j, sl]
            e16 = exb[j, sl]
            h16 = hwb[j, sl]
            sg = plsc.load_gather(sN, [i16])
            gg = plsc.load_gather(gN, [i16])
            abuf[sl] = e16 * h16 / (gg + 1e-5 * sg + 1e-21)
        base = (w * ROWS_PER + j) * 128
        pltpu.sync_copy(abuf, alpha.at[pl.ds(base, 128)])
        return carry

    lax.fori_loop(0, _nrows(w), row, 0)


def _sc_alpha(ex2, hw2, ix2, ssum, shw):
    fn = pl.kernel(
        _alpha_body,
        out_type=jax.ShapeDtypeStruct((E_PAD,), F32),
        mesh=_MESH,
        scratch_types=[pltpu.VMEM((ROWS_PER, 128), F32),
                       pltpu.VMEM((ROWS_PER, 128), F32),
                       pltpu.VMEM((ROWS_PER, 128), jnp.int32),
                       pltpu.VMEM((N,), F32),
                       pltpu.VMEM((N,), F32),
                       pltpu.VMEM((128,), F32)],
        compiler_params=_SC_PARAMS,
    )
    return fn(ex2, hw2, ix2, ssum, shw)


def _scatter_body(m, ix2, acc, ixb, ixt0, ixt1, mb0, mb1,
                  semr0, semr1, sems0, sems1, spacc):
    c = lax.axis_index("c")
    s = lax.axis_index("s")

    pltpu.sync_copy(ix2.at[pl.ds(s * SROWS, SROWS)], ixb)
    nrows = jnp.clip(R - s * SROWS, 0, SROWS).astype(jnp.int32)
    lo = c * NHALF
    mbs = (mb0, mb1)
    ixts = (ixt0, ixt1)
    rsems = (semr0, semr1)
    ssems = (sems0, sems1)
    zero16 = jnp.zeros((16,), F32)

    def zm(i, carry):
        for k in range(8):
            mb0[i, pl.ds(k * 16, 16)] = zero16
        return carry

    lax.fori_loop(0, 128, zm, 0)
    pltpu.sync_copy(mb0, spacc.at[pl.ds(s * 328, 128)])
    pltpu.sync_copy(mb0, spacc.at[pl.ds(s * 328 + 128, 128)])
    pltpu.sync_copy(mb0.at[pl.ds(0, 72)],
                    spacc.at[pl.ds(s * 328 + 256, 72)])
    plsc.subcore_barrier()

    def pair(j2, carry):
        js = (2 * j2, 2 * j2 + 1)
        for b in range(2):
            @pl.when(js[b] < nrows)
            def _(b=b):
                base = (s * SROWS + js[b]) * 128
                pltpu.async_copy(m.at[pl.ds(base, 128)], mbs[b], rsems[b])
        for b in range(2):
            @pl.when(js[b] < nrows)
            def _(b=b):
                for k in range(8):
                    sl = pl.ds(k * 16, 16)
                    i16 = ixb[js[b], sl]
                    iloc = i16 - lo
                    ok = (iloc >= 0) & (iloc < NHALF)
                    ixts[b][sl] = jnp.where(ok, iloc, NHALF + 120)
                base = (s * SROWS + js[b]) * 128
                pltpu.make_async_copy(
                    m.at[pl.ds(base, 128)], mbs[b], rsems[b]).wait()
                pltpu.async_copy(mbs[b], spacc.at[ixts[b]], ssems[b],
                                 add=True)
        for b in range(2):
            @pl.when(js[b] < nrows)
            def _(b=b):
                pltpu.make_async_copy(mbs[b], spacc.at[ixts[b]],
                                      ssems[b]).wait()
        return carry

    lax.fori_loop(0, SROWS // 2, pair, 0)
    plsc.subcore_barrier()
    for q in range(2):
        start = s * 328 + q * 128
        pltpu.sync_copy(spacc.at[pl.ds(start, 128)], mb0)
        pltpu.sync_copy(mb0, acc.at[c, pl.ds(start, 128)])
    start = s * 328 + 256
    pltpu.sync_copy(spacc.at[pl.ds(start, 72)], mb0.at[pl.ds(0, 72)])
    pltpu.sync_copy(mb0.at[pl.ds(0, 72)], acc.at[c, pl.ds(start, 72)])


def _sc_scatter(m, ix2):
    fn = pl.kernel(
        _scatter_body,
        out_type=jax.ShapeDtypeStruct((2, ACC_ROWS, H), F32),
        mesh=_MESH,
        scratch_types=[pltpu.VMEM((SROWS, 128), jnp.int32),
                       pltpu.VMEM((128,), jnp.int32),
                       pltpu.VMEM((128,), jnp.int32),
                       pltpu.VMEM((128, H), F32),
                       pltpu.VMEM((128, H), F32),
                       pltpu.SemaphoreType.DMA,
                       pltpu.SemaphoreType.DMA,
                       pltpu.SemaphoreType.DMA,
                       pltpu.SemaphoreType.DMA,
                       pltpu.VMEM_SHARED((ACC_ROWS, H), F32)],
        compiler_params=_SC_PARAMS,
    )
    return fn(m, ix2)


def _tri_stage(x_tables, hi2d, ha, hw2, uv, p, pre):
    t, ti, tj, tk, tr = x_tables
    i0, i1, i2 = hi2d
    g0, g1, g2, xrg = _sc_gather(uv, i0, i1, i2, ti, tj, tk, tr)
    m, ex = _tc_msg(g0, g1, g2, xrg, ha, p, pre)
    ex2 = ex.reshape(R_PAD, 128)
    ix2 = (i0, i1, i2)[uv]
    parts = _sc_sums(ex2, hw2, ix2)
    sp = parts[:NW * N].reshape(NW, N)
    gp = parts[NW * N:].reshape(NW, N)
    ssum, shw = _tc_reduce(sp, gp)
    alpha = _sc_alpha(ex2, hw2, ix2, ssum, shw)
    msc = _tc_scale(m, alpha.reshape(E_PAD, 1))
    acc = _sc_scatter(msc, ix2)
    return t, acc


def kernel(x, hyperedge_index, hyperedge_attr, hyperedge_weight, params):
    p = params
    hi = hyperedge_index.astype(jnp.int32)
    pad2d = lambda a: jnp.pad(a.reshape(R, 128), ((0, R_PAD - R), (0, 0)))
    hi2d = tuple(pad2d(hi[v]) for v in range(3))
    hw2 = pad2d(hyperedge_weight)

    t0, xi, xj, xk, xr = _tc_prep0(x, p)
    t0, acc1 = _tri_stage((t0, xi, xj, xk, xr), hi2d, hyperedge_attr,
                          hw2, 2, p, "outp")
    x1, yi, yj, yk, yr = _tc_update1(t0, acc1, p)
    x1, acc2 = _tri_stage((x1, yi, yj, yk, yr), hi2d, hyperedge_attr,
                          hw2, 0, p, "inp")
    return _tc_update2(x1, acc2, p)


# speedup vs baseline: 1.3708x; 1.1115x over previous
import functools

import jax
import jax.numpy as jnp
from jax import lax
from jax.experimental import pallas as pl
from jax.experimental.pallas import tpu as pltpu
from jax.experimental.pallas import tpu_sc as plsc

N = 10000
E = 160000
D = 256
H = 128
DE = 16

NW = 32
R = E // 128
ROWS_PER = 40
R_PAD = NW * ROWS_PER
E_PAD = R_PAD * 128
NHALF = N // 2
SROWS = R_PAD // 16
ACC_ROWS = 5248

F32 = jnp.float32
BF16 = jnp.bfloat16


def _elu(v):
    return jnp.where(v > 0, v, jnp.exp(jnp.minimum(v, 0.0)) - 1.0)


def _lrelu(v):
    return jnp.where(v > 0, v, 0.01 * v)


def _dot(a, b):
    return jax.lax.dot_general(a, b, (((1,), (0,)), ((), ())),
                               preferred_element_type=F32)


def _pack2(a, b):
    ai = lax.bitcast_convert_type(a.astype(BF16), jnp.int16)
    bi_ = lax.bitcast_convert_type(b.astype(BF16), jnp.int16)
    return ((ai.astype(jnp.int32) & jnp.int32(0xFFFF)) |
            (bi_.astype(jnp.int32) << 16))


def _unpack_lo(g):
    return lax.bitcast_convert_type(g << 16, F32)


def _unpack_hi(g):
    return lax.bitcast_convert_type(g & jnp.int32(-65536), F32)


def _transfer(x, w1, w2):
    out = _elu(_dot(x, w1.T))
    mu = jnp.mean(out, axis=-1, keepdims=True)
    var = jnp.var(out, axis=-1, keepdims=True)
    out = (out - mu) / jnp.sqrt(var + 1e-5)
    return _elu(_dot(out, w2.T) + x)


def _prep0_body(x, w1, w2, wi, bi, wj, bj, wk, bk, wr,
                t0, xi, xj, tkr):
    t = _transfer(x[...], w1[...], w2[...])
    t0[...] = t
    xi[...] = _dot(t, wi[...].T) + bi[...]
    xj[...] = _dot(t, wj[...].T) + bj[...]
    xk = _dot(t, wk[...].T) + bk[...]
    xr = _dot(t, wr[...].T)
    tkr[...] = _pack2(xk, xr)


def _msg_body(uv, g0, g1, g2, ha, we, be, wl, att, m_out, ex_out):
    e = _dot(ha[...], we[...].T) + be[...]
    gs = [g0[...], g1[...], g2[...]]
    xrg = _unpack_hi(gs[uv])
    gs[uv] = _unpack_lo(gs[uv])
    m = _elu(gs[0] + gs[1] + gs[2] + e)
    m_out[...] = m.astype(BF16)
    q = _lrelu(_dot(m, wl[...].T) + xrg)
    s = jnp.sum(q * att[...], axis=1, keepdims=True)
    ex_out[...] = jnp.exp(_lrelu(s))


def _reduce_body(sp, gp, ssum, shw):
    ssum[...] = jnp.sum(sp[...], axis=0)
    shw[...] = jnp.sum(gp[...], axis=0)


def _scale_body(m, al, out):
    out[...] = m[...].astype(F32) * al[...]


def _update1_body(xp, acc, w2, cb, wi, bi, wj, bj, wk, bk, wr,
                  x1, tir, yj, yk):
    a = acc[0]
    t = _elu(xp[...] + _dot(a, w2[...].T) + cb[...])
    x1[...] = t
    xi = _dot(t, wi[...].T) + bi[...]
    xr = _dot(t, wr[...].T)
    tir[...] = _pack2(xi, xr)
    yj[...] = _dot(t, wj[...].T) + bj[...]
    yk[...] = _dot(t, wk[...].T) + bk[...]


def _update2_body(xp, acc, w2, cb, tw1, tw2, ew, out):
    a = acc[0]
    t = _elu(xp[...] + _dot(a, w2[...].T) + cb[...])
    t = _transfer(t, tw1[...], tw2[...])
    out[...] = _dot(t, ew[...].T)


_BN = 1000
_BE = 6400


def _full(shape):
    return pl.BlockSpec(shape, lambda i: tuple(0 for _ in shape))


def _tc_prep0(x, p):
    grid = (N // _BN,)
    return pl.pallas_call(
        _prep0_body,
        grid=grid,
        in_specs=[pl.BlockSpec((_BN, D), lambda i: (i, 0)),
                  _full((D, D)), _full((D, D)),
                  _full((H, D)), _full((1, H)),
                  _full((H, D)), _full((1, H)),
                  _full((H, D)), _full((1, H)),
                  _full((H, D))],
        out_specs=[pl.BlockSpec((_BN, D), lambda i: (i, 0))] +
                  [pl.BlockSpec((_BN, H), lambda i: (i, 0))] * 3,
        out_shape=[jax.ShapeDtypeStruct((N, D), F32),
                   jax.ShapeDtypeStruct((N, H), F32),
                   jax.ShapeDtypeStruct((N, H), F32),
                   jax.ShapeDtypeStruct((N, H), jnp.int32)],
    )(x, p["trans0_W1"], p["trans0_W2"],
      p["outp_Wi"], p["outp_bi"].reshape(1, H),
      p["outp_Wj"], p["outp_bj"].reshape(1, H),
      p["outp_Wk"], p["outp_bk"].reshape(1, H),
      p["outp_Wr"])


def _tc_msg(uv, g0, g1, g2, ha, p, pre):
    grid = (E // _BE,)
    return pl.pallas_call(
        functools.partial(_msg_body, uv),
        grid=grid,
        in_specs=[pl.BlockSpec((_BE, H), lambda i: (i, 0))] * 3 +
                 [pl.BlockSpec((_BE, DE), lambda i: (i, 0)),
                  _full((H, DE)), _full((1, H)), _full((H, H)),
                  _full((1, H))],
        out_specs=[pl.BlockSpec((_BE, H), lambda i: (i, 0)),
                   pl.BlockSpec((_BE, 1), lambda i: (i, 0))],
        out_shape=[jax.ShapeDtypeStruct((E_PAD, H), BF16),
                   jax.ShapeDtypeStruct((E_PAD, 1), F32)],
    )(g0, g1, g2, ha,
      p[pre + "_We"], p[pre + "_be"].reshape(1, H),
      p[pre + "_Wl"], p[pre + "_att"].reshape(1, H))


def _tc_reduce(sp, gp):
    return pl.pallas_call(
        _reduce_body,
        out_shape=[jax.ShapeDtypeStruct((N,), F32)] * 2,
    )(sp, gp)


def _tc_scale(m, al):
    grid = (E // _BE,)
    return pl.pallas_call(
        _scale_body,
        grid=grid,
        in_specs=[pl.BlockSpec((_BE, H), lambda i: (i, 0)),
                  pl.BlockSpec((_BE, 1), lambda i: (i, 0))],
        out_specs=pl.BlockSpec((_BE, H), lambda i: (i, 0)),
        out_shape=jax.ShapeDtypeStruct((E_PAD, H), F32),
    )(m, al)


def _tc_update1(xp, acc, p):
    grid = (N // _BN,)
    return pl.pallas_call(
        _update1_body,
        grid=grid,
        in_specs=[pl.BlockSpec((_BN, D), lambda i: (i, 0)),
                  pl.BlockSpec((1, _BN, H), lambda i: (i // 5, i % 5, 0)),
                  _full((D, H)), _full((1, D)),
                  _full((H, D)), _full((1, H)),
                  _full((H, D)), _full((1, H)),
                  _full((H, D)), _full((1, H)),
                  _full((H, D))],
        out_specs=[pl.BlockSpec((_BN, D), lambda i: (i, 0))] +
                  [pl.BlockSpec((_BN, H), lambda i: (i, 0))] * 3,
        out_shape=[jax.ShapeDtypeStruct((N, D), F32),
                   jax.ShapeDtypeStruct((N, H), jnp.int32),
                   jax.ShapeDtypeStruct((N, H), F32),
                   jax.ShapeDtypeStruct((N, H), F32)],
    )(xp, acc, p["outp_W2"], p["outp_cbias"].reshape(1, D),
      p["inp_Wi"], p["inp_bi"].reshape(1, H),
      p["inp_Wj"], p["inp_bj"].reshape(1, H),
      p["inp_Wk"], p["inp_bk"].reshape(1, H),
      p["inp_Wr"])


def _tc_update2(xp, acc, p):
    grid = (N // _BN,)
    return pl.pallas_call(
        _update2_body,
        grid=grid,
        in_specs=[pl.BlockSpec((_BN, D), lambda i: (i, 0)),
                  pl.BlockSpec((1, _BN, H), lambda i: (i // 5, i % 5, 0)),
                  _full((D, H)), _full((1, D)),
                  _full((D, D)), _full((D, D)), _full((D, D))],
        out_specs=pl.BlockSpec((_BN, D), lambda i: (i, 0)),
        out_shape=jax.ShapeDtypeStruct((N, D), F32),
    )(xp, acc, p["inp_W2"], p["inp_cbias"].reshape(1, D),
      p["trans1_W1"], p["trans1_W2"], p["exit_W"])


_MESH = plsc.VectorSubcoreMesh(core_axis_name="c", subcore_axis_name="s")
_SC_PARAMS = pltpu.CompilerParams(needs_layout_passes=False)


def _wid():
    return lax.axis_index("s") * 2 + lax.axis_index("c")


def _stage_rows(src2d, dst, w):
    pltpu.sync_copy(src2d.at[pl.ds(w * ROWS_PER, ROWS_PER)], dst)


def _nrows(w):
    return jnp.clip(R - w * ROWS_PER, 0, ROWS_PER).astype(jnp.int32)


def _gather_body(uv, i0, i1, i2, ta, tb, tc,
                 g0, g1, g2,
                 i0b, i1b, i2b,
                 ba0, ba1, ba2, bb0, bb1, bb2,
                 semga, semgb, semwa, semwb):
    w = _wid()
    _stage_rows(i0, i0b, w)
    _stage_rows(i1, i1b, w)
    _stage_rows(i2, i2b, w)
    ncc = _nrows(w) * 2
    bufs = ((ba0, ba1, ba2), (bb0, bb1, bb2))
    gsems = (semga, semgb)
    wsems = (semwa, semwb)
    outs = (g0, g1, g2)
    tabs = (ta, tb, tc)
    idxb = (i0b, i1b, i2b)

    def srcs(cc):
        j = cc // 2
        sl = pl.ds((cc % 2) * 64, 64)
        return tuple(t_.at[ib.at[j, sl]] for t_, ib in zip(tabs, idxb))

    def pair(j2, carry):
        js = (2 * j2, 2 * j2 + 1)
        for b in range(2):
            @pl.when(js[b] < ncc)
            def _(b=b):
                for s_, b_ in zip(srcs(js[b]), bufs[b]):
                    pltpu.async_copy(s_, b_, gsems[b])
        for b in range(2):
            @pl.when(js[b] < ncc)
            def _(b=b):
                for s_, b_ in zip(srcs(js[b]), bufs[b]):
                    pltpu.make_async_copy(s_, b_, gsems[b]).wait()
                base = w * ROWS_PER * 128 + js[b] * 64
                for o_, b_ in zip(outs, bufs[b]):
                    pltpu.async_copy(b_, o_.at[pl.ds(base, 64)], wsems[b])
        for b in range(2):
            @pl.when(js[b] < ncc)
            def _(b=b):
                base = w * ROWS_PER * 128 + js[b] * 64
                for o_, b_ in zip(outs, bufs[b]):
                    pltpu.make_async_copy(
                        b_, o_.at[pl.ds(base, 64)], wsems[b]).wait()
        return carry

    lax.fori_loop(0, ROWS_PER, pair, 0)


def _sc_gather(uv, i0, i1, i2, ta, tb, tc):
    dts = [jnp.int32 if v == uv else F32 for v in range(3)]
    fn = pl.kernel(
        functools.partial(_gather_body, uv),
        out_type=[jax.ShapeDtypeStruct((E_PAD, H), dt) for dt in dts],
        mesh=_MESH,
        scratch_types=[pltpu.VMEM((ROWS_PER, 128), jnp.int32)] * 3 +
                      [pltpu.VMEM((64, H), dt) for dt in dts] * 2 +
                      [pltpu.SemaphoreType.DMA] * 4,
        compiler_params=_SC_PARAMS,
    )
    return fn(i0, i1, i2, ta, tb, tc)


def _sums_body(ex2, hw2, ix2, out, exb, hwb, ixb, s1, s2):
    w = _wid()
    zero16 = jnp.zeros((16,), F32)

    def z(i, carry):
        s1[pl.ds(i * 16, 16)] = zero16
        s2[pl.ds(i * 16, 16)] = zero16
        return carry

    lax.fori_loop(0, N // 16, z, 0)
    _stage_rows(ex2, exb, w)
    _stage_rows(hw2, hwb, w)
    _stage_rows(ix2, ixb, w)

    def row(j, carry):
        for k in range(8):
            sl = pl.ds(k * 16, 16)
            i16 = ixb[j, sl]
            e16 = exb[j, sl]
            h16 = hwb[j, sl]
            plsc.addupdate_scatter(s1, [i16], e16)
            plsc.addupdate_scatter(s2, [i16], e16 * h16)
        return carry

    lax.fori_loop(0, _nrows(w), row, 0)
    pltpu.sync_copy(s1, out.at[pl.ds(w * N, N)])
    pltpu.sync_copy(s2, out.at[pl.ds((NW + w) * N, N)])


def _sc_sums(ex2, hw2, ix2):
    fn = pl.kernel(
        _sums_body,
        out_type=jax.ShapeDtypeStruct((2 * NW * N,), F32),
        mesh=_MESH,
        scratch_types=[pltpu.VMEM((ROWS_PER, 128), F32),
                       pltpu.VMEM((ROWS_PER, 128), F32),
                       pltpu.VMEM((ROWS_PER, 128), jnp.int32),
                       pltpu.VMEM((N,), F32),
                       pltpu.VMEM((N,), F32)],
        compiler_params=_SC_PARAMS,
    )
    return fn(ex2, hw2, ix2)


def _alpha_body(ex2, hw2, ix2, ssum, shw, alpha,
                exb, hwb, ixb, sN, gN, abuf):
    w = _wid()
    pltpu.sync_copy(ssum, sN)
    pltpu.sync_copy(shw, gN)
    _stage_rows(ex2, exb, w)
    _stage_rows(hw2, hwb, w)
    _stage_rows(ix2, ixb, w)

    def row(j, carry):
        for k in range(8):
            sl = pl.ds(k * 16, 16)
            i16 = ixb[j, sl]
            e16 = exb[j, sl]
            h16 = hwb[j, sl]
            sg = plsc.load_gather(sN, [i16])
            gg = plsc.load_gather(gN, [i16])
            abuf[sl] = e16 * h16 / (gg + 1e-5 * sg + 1e-21)
        base = (w * ROWS_PER + j) * 128
        pltpu.sync_copy(abuf, alpha.at[pl.ds(base, 128)])
        return carry

    lax.fori_loop(0, _nrows(w), row, 0)


def _sc_alpha(ex2, hw2, ix2, ssum, shw):
    fn = pl.kernel(
        _alpha_body,
        out_type=jax.ShapeDtypeStruct((E_PAD,), F32),
        mesh=_MESH,
        scratch_types=[pltpu.VMEM((ROWS_PER, 128), F32),
                       pltpu.VMEM((ROWS_PER, 128), F32),
                       pltpu.VMEM((ROWS_PER, 128), jnp.int32),
                       pltpu.VMEM((N,), F32),
                       pltpu.VMEM((N,), F32),
                       pltpu.VMEM((128,), F32)],
        compiler_params=_SC_PARAMS,
    )
    return fn(ex2, hw2, ix2, ssum, shw)


def _scatter_body(m, ix2, acc, ixb, ixt0, ixt1, mb0, mb1,
                  semr0, semr1, sems0, sems1, spacc):
    c = lax.axis_index("c")
    s = lax.axis_index("s")

    pltpu.sync_copy(ix2.at[pl.ds(s * SROWS, SROWS)], ixb)
    nrows = jnp.clip(R - s * SROWS, 0, SROWS).astype(jnp.int32)
    lo = c * NHALF
    mbs = (mb0, mb1)
    ixts = (ixt0, ixt1)
    rsems = (semr0, semr1)
    ssems = (sems0, sems1)
    zero16 = jnp.zeros((16,), F32)

    def zm(i, carry):
        for k in range(8):
            mb0[i, pl.ds(k * 16, 16)] = zero16
        return carry

    lax.fori_loop(0, 128, zm, 0)
    pltpu.sync_copy(mb0, spacc.at[pl.ds(s * 328, 128)])
    pltpu.sync_copy(mb0, spacc.at[pl.ds(s * 328 + 128, 128)])
    pltpu.sync_copy(mb0.at[pl.ds(0, 72)],
                    spacc.at[pl.ds(s * 328 + 256, 72)])
    plsc.subcore_barrier()

    def pair(j2, carry):
        js = (2 * j2, 2 * j2 + 1)
        for b in range(2):
            @pl.when(js[b] < nrows)
            def _(b=b):
                base = (s * SROWS + js[b]) * 128
                pltpu.async_copy(m.at[pl.ds(base, 128)], mbs[b], rsems[b])
        for b in range(2):
            @pl.when(js[b] < nrows)
            def _(b=b):
                for k in range(8):
                    sl = pl.ds(k * 16, 16)
                    i16 = ixb[js[b], sl]
                    iloc = i16 - lo
                    ok = (iloc >= 0) & (iloc < NHALF)
                    ixts[b][sl] = jnp.where(ok, iloc, NHALF + 120)
                base = (s * SROWS + js[b]) * 128
                pltpu.make_async_copy(
                    m.at[pl.ds(base, 128)], mbs[b], rsems[b]).wait()
                pltpu.async_copy(mbs[b], spacc.at[ixts[b]], ssems[b],
                                 add=True)
        for b in range(2):
            @pl.when(js[b] < nrows)
            def _(b=b):
                pltpu.make_async_copy(mbs[b], spacc.at[ixts[b]],
                                      ssems[b]).wait()
        return carry

    lax.fori_loop(0, SROWS // 2, pair, 0)
    plsc.subcore_barrier()
    for q in range(2):
        start = s * 328 + q * 128
        pltpu.sync_copy(spacc.at[pl.ds(start, 128)], mb0)
        pltpu.sync_copy(mb0, acc.at[c, pl.ds(start, 128)])
    start = s * 328 + 256
    pltpu.sync_copy(spacc.at[pl.ds(start, 72)], mb0.at[pl.ds(0, 72)])
    pltpu.sync_copy(mb0.at[pl.ds(0, 72)], acc.at[c, pl.ds(start, 72)])


def _sc_scatter(m, ix2):
    fn = pl.kernel(
        _scatter_body,
        out_type=jax.ShapeDtypeStruct((2, ACC_ROWS, H), F32),
        mesh=_MESH,
        scratch_types=[pltpu.VMEM((SROWS, 128), jnp.int32),
                       pltpu.VMEM((128,), jnp.int32),
                       pltpu.VMEM((128,), jnp.int32),
                       pltpu.VMEM((128, H), F32),
                       pltpu.VMEM((128, H), F32),
                       pltpu.SemaphoreType.DMA,
                       pltpu.SemaphoreType.DMA,
                       pltpu.SemaphoreType.DMA,
                       pltpu.SemaphoreType.DMA,
                       pltpu.VMEM_SHARED((ACC_ROWS, H), F32)],
        compiler_params=_SC_PARAMS,
    )
    return fn(m, ix2)


def _tri_stage(x_tables, hi2d, ha, hw2, uv, p, pre):
    t, ta, tb, tc = x_tables
    i0, i1, i2 = hi2d
    g0, g1, g2 = _sc_gather(uv, i0, i1, i2, ta, tb, tc)
    m, ex = _tc_msg(uv, g0, g1, g2, ha, p, pre)
    ex2 = ex.reshape(R_PAD, 128)
    ix2 = (i0, i1, i2)[uv]
    parts = _sc_sums(ex2, hw2, ix2)
    sp = parts[:NW * N].reshape(NW, N)
    gp = parts[NW * N:].reshape(NW, N)
    ssum, shw = _tc_reduce(sp, gp)
    alpha = _sc_alpha(ex2, hw2, ix2, ssum, shw)
    msc = _tc_scale(m, alpha.reshape(E_PAD, 1))
    acc = _sc_scatter(msc, ix2)
    return t, acc


def kernel(x, hyperedge_index, hyperedge_attr, hyperedge_weight, params):
    p = params
    hi = hyperedge_index.astype(jnp.int32)
    pad2d = lambda a: jnp.pad(a.reshape(R, 128), ((0, R_PAD - R), (0, 0)))
    hi2d = tuple(pad2d(hi[v]) for v in range(3))
    hw2 = pad2d(hyperedge_weight)

    t0, xi, xj, tkr = _tc_prep0(x, p)
    t0, acc1 = _tri_stage((t0, xi, xj, tkr), hi2d, hyperedge_attr,
                          hw2, 2, p, "outp")
    x1, tir, yj, yk = _tc_update1(t0, acc1, p)
    x1, acc2 = _tri_stage((x1, tir, yj, yk), hi2d, hyperedge_attr,
                          hw2, 0, p, "inp")
    return _tc_update2(x1, acc2, p)


# speedup vs baseline: 1.3844x; 1.0100x over previous
import functools

import jax
import jax.numpy as jnp
from jax import lax
from jax.experimental import pallas as pl
from jax.experimental.pallas import tpu as pltpu
from jax.experimental.pallas import tpu_sc as plsc

N = 10000
E = 160000
D = 256
H = 128
DE = 16

NW = 32
R = E // 128
ROWS_PER = 40
R_PAD = NW * ROWS_PER
E_PAD = R_PAD * 128
NHALF = N // 2
SROWS = R_PAD // 16
ACC_ROWS = 5248

F32 = jnp.float32
BF16 = jnp.bfloat16


def _elu(v):
    return jnp.where(v > 0, v, jnp.exp(jnp.minimum(v, 0.0)) - 1.0)


def _lrelu(v):
    return jnp.where(v > 0, v, 0.01 * v)


def _dot(a, b):
    return jax.lax.dot_general(a, b, (((1,), (0,)), ((), ())),
                               preferred_element_type=F32)


def _pack2(a, b):
    ai = lax.bitcast_convert_type(a.astype(BF16), jnp.int16)
    bi_ = lax.bitcast_convert_type(b.astype(BF16), jnp.int16)
    return ((ai.astype(jnp.int32) & jnp.int32(0xFFFF)) |
            (bi_.astype(jnp.int32) << 16))


def _unpack_lo(g):
    return lax.bitcast_convert_type(g << 16, F32)


def _unpack_hi(g):
    return lax.bitcast_convert_type(g & jnp.int32(-65536), F32)


def _transfer(x, w1, w2):
    out = _elu(_dot(x, w1.T))
    mu = jnp.mean(out, axis=-1, keepdims=True)
    var = jnp.var(out, axis=-1, keepdims=True)
    out = (out - mu) / jnp.sqrt(var + 1e-5)
    return _elu(_dot(out, w2.T) + x)


def _prep0_body(x, w1, w2, wi, bi, wj, bj, wk, bk, wr,
                t0, xi, xj, tkr):
    t = _transfer(x[...], w1[...], w2[...])
    t0[...] = t
    xi[...] = _dot(t, wi[...].T) + bi[...]
    xj[...] = _dot(t, wj[...].T) + bj[...]
    xk = _dot(t, wk[...].T) + bk[...]
    xr = _dot(t, wr[...].T)
    tkr[...] = _pack2(xk, xr)


def _msg_body(uv, g0, g1, g2, ha, we, be, wl, att, m_out, ex_out):
    e = _dot(ha[...], we[...].T) + be[...]
    gs = [g0[...], g1[...], g2[...]]
    xrg = _unpack_hi(gs[uv])
    gs[uv] = _unpack_lo(gs[uv])
    m = _elu(gs[0] + gs[1] + gs[2] + e)
    m_out[...] = m.astype(BF16)
    q = _lrelu(_dot(m, wl[...].T) + xrg)
    s = jnp.sum(q * att[...], axis=1, keepdims=True)
    ex_out[...] = jnp.exp(_lrelu(s))


def _reduce_body(sp, gp, ssum, shw):
    ssum[...] = jnp.sum(sp[...], axis=0)
    shw[...] = jnp.sum(gp[...], axis=0)


def _scale_body(m, al, out):
    out[...] = m[...].astype(F32) * al[...]


def _update1_body(xp, acc, w2, cb, wi, bi, wj, bj, wk, bk, wr,
                  x1, tir, yj, yk):
    a = acc[0]
    t = _elu(xp[...] + _dot(a, w2[...].T) + cb[...])
    x1[...] = t
    xi = _dot(t, wi[...].T) + bi[...]
    xr = _dot(t, wr[...].T)
    tir[...] = _pack2(xi, xr)
    yj[...] = _dot(t, wj[...].T) + bj[...]
    yk[...] = _dot(t, wk[...].T) + bk[...]


def _update2_body(xp, acc, w2, cb, tw1, tw2, ew, out):
    a = acc[0]
    t = _elu(xp[...] + _dot(a, w2[...].T) + cb[...])
    t = _transfer(t, tw1[...], tw2[...])
    out[...] = _dot(t, ew[...].T)


_BN = 1000
_BE = 6400


def _full(shape):
    return pl.BlockSpec(shape, lambda i: tuple(0 for _ in shape))


def _tc_prep0(x, p):
    grid = (N // _BN,)
    return pl.pallas_call(
        _prep0_body,
        grid=grid,
        in_specs=[pl.BlockSpec((_BN, D), lambda i: (i, 0)),
                  _full((D, D)), _full((D, D)),
                  _full((H, D)), _full((1, H)),
                  _full((H, D)), _full((1, H)),
                  _full((H, D)), _full((1, H)),
                  _full((H, D))],
        out_specs=[pl.BlockSpec((_BN, D), lambda i: (i, 0))] +
                  [pl.BlockSpec((_BN, H), lambda i: (i, 0))] * 3,
        out_shape=[jax.ShapeDtypeStruct((N, D), F32),
                   jax.ShapeDtypeStruct((N, H), F32),
                   jax.ShapeDtypeStruct((N, H), F32),
                   jax.ShapeDtypeStruct((N, H), jnp.int32)],
    )(x, p["trans0_W1"], p["trans0_W2"],
      p["outp_Wi"], p["outp_bi"].reshape(1, H),
      p["outp_Wj"], p["outp_bj"].reshape(1, H),
      p["outp_Wk"], p["outp_bk"].reshape(1, H),
      p["outp_Wr"])


def _tc_msg(uv, g0, g1, g2, ha, p, pre):
    grid = (E // _BE,)
    return pl.pallas_call(
        functools.partial(_msg_body, uv),
        grid=grid,
        in_specs=[pl.BlockSpec((_BE, H), lambda i: (i, 0))] * 3 +
                 [pl.BlockSpec((_BE, DE), lambda i: (i, 0)),
                  _full((H, DE)), _full((1, H)), _full((H, H)),
                  _full((1, H))],
        out_specs=[pl.BlockSpec((_BE, H), lambda i: (i, 0)),
                   pl.BlockSpec((_BE, 1), lambda i: (i, 0))],
        out_shape=[jax.ShapeDtypeStruct((E_PAD, H), BF16),
                   jax.ShapeDtypeStruct((E_PAD, 1), F32)],
    )(g0, g1, g2, ha,
      p[pre + "_We"], p[pre + "_be"].reshape(1, H),
      p[pre + "_Wl"], p[pre + "_att"].reshape(1, H))


def _tc_reduce(sp, gp):
    return pl.pallas_call(
        _reduce_body,
        out_shape=[jax.ShapeDtypeStruct((N,), F32)] * 2,
    )(sp, gp)


def _tc_scale(m, al):
    grid = (E // _BE,)
    return pl.pallas_call(
        _scale_body,
        grid=grid,
        in_specs=[pl.BlockSpec((_BE, H), lambda i: (i, 0)),
                  pl.BlockSpec((_BE, 1), lambda i: (i, 0))],
        out_specs=pl.BlockSpec((_BE, H), lambda i: (i, 0)),
        out_shape=jax.ShapeDtypeStruct((E_PAD, H), F32),
    )(m, al)


def _tc_update1(xp, acc, p):
    grid = (N // _BN,)
    return pl.pallas_call(
        _update1_body,
        grid=grid,
        in_specs=[pl.BlockSpec((_BN, D), lambda i: (i, 0)),
                  pl.BlockSpec((1, _BN, H), lambda i: (i // 5, i % 5, 0)),
                  _full((D, H)), _full((1, D)),
                  _full((H, D)), _full((1, H)),
                  _full((H, D)), _full((1, H)),
                  _full((H, D)), _full((1, H)),
                  _full((H, D))],
        out_specs=[pl.BlockSpec((_BN, D), lambda i: (i, 0))] +
                  [pl.BlockSpec((_BN, H), lambda i: (i, 0))] * 3,
        out_shape=[jax.ShapeDtypeStruct((N, D), F32),
                   jax.ShapeDtypeStruct((N, H), jnp.int32),
                   jax.ShapeDtypeStruct((N, H), F32),
                   jax.ShapeDtypeStruct((N, H), F32)],
    )(xp, acc, p["outp_W2"], p["outp_cbias"].reshape(1, D),
      p["inp_Wi"], p["inp_bi"].reshape(1, H),
      p["inp_Wj"], p["inp_bj"].reshape(1, H),
      p["inp_Wk"], p["inp_bk"].reshape(1, H),
      p["inp_Wr"])


def _tc_update2(xp, acc, p):
    grid = (N // _BN,)
    return pl.pallas_call(
        _update2_body,
        grid=grid,
        in_specs=[pl.BlockSpec((_BN, D), lambda i: (i, 0)),
                  pl.BlockSpec((1, _BN, H), lambda i: (i // 5, i % 5, 0)),
                  _full((D, H)), _full((1, D)),
                  _full((D, D)), _full((D, D)), _full((D, D))],
        out_specs=pl.BlockSpec((_BN, D), lambda i: (i, 0)),
        out_shape=jax.ShapeDtypeStruct((N, D), F32),
    )(xp, acc, p["inp_W2"], p["inp_cbias"].reshape(1, D),
      p["trans1_W1"], p["trans1_W2"], p["exit_W"])


_MESH = plsc.VectorSubcoreMesh(core_axis_name="c", subcore_axis_name="s")
_SC_PARAMS = pltpu.CompilerParams(needs_layout_passes=False)


def _wid():
    return lax.axis_index("s") * 2 + lax.axis_index("c")


def _stage_rows(src2d, dst, w):
    pltpu.sync_copy(src2d.at[pl.ds(w * ROWS_PER, ROWS_PER)], dst)


def _nrows(w):
    return jnp.clip(R - w * ROWS_PER, 0, ROWS_PER).astype(jnp.int32)


def _gather_body(uv, i0, i1, i2, ta, tb, tc,
                 g0, g1, g2,
                 i0b, i1b, i2b,
                 ba0, ba1, ba2, bb0, bb1, bb2,
                 bc0, bc1, bc2, bd0, bd1, bd2,
                 sga, sgb, sgc, sgd, swa, swb, swc, swd):
    w = _wid()
    _stage_rows(i0, i0b, w)
    _stage_rows(i1, i1b, w)
    _stage_rows(i2, i2b, w)
    ncc = _nrows(w) * 2
    bufs = ((ba0, ba1, ba2), (bb0, bb1, bb2),
            (bc0, bc1, bc2), (bd0, bd1, bd2))
    gsems = (sga, sgb, sgc, sgd)
    wsems = (swa, swb, swc, swd)
    outs = (g0, g1, g2)
    tabs = (ta, tb, tc)
    idxb = (i0b, i1b, i2b)

    def srcs(cc):
        j = cc // 2
        sl = pl.ds((cc % 2) * 64, 64)
        return tuple(t_.at[ib.at[j, sl]] for t_, ib in zip(tabs, idxb))

    def quad(j2, carry):
        js = tuple(4 * j2 + b for b in range(4))
        for b in range(4):
            @pl.when(js[b] < ncc)
            def _(b=b):
                for s_, b_ in zip(srcs(js[b]), bufs[b]):
                    pltpu.async_copy(s_, b_, gsems[b])
        for b in range(4):
            @pl.when(js[b] < ncc)
            def _(b=b):
                for s_, b_ in zip(srcs(js[b]), bufs[b]):
                    pltpu.make_async_copy(s_, b_, gsems[b]).wait()
                base = w * ROWS_PER * 128 + js[b] * 64
                for o_, b_ in zip(outs, bufs[b]):
                    pltpu.async_copy(b_, o_.at[pl.ds(base, 64)], wsems[b])
        for b in range(4):
            @pl.when(js[b] < ncc)
            def _(b=b):
                base = w * ROWS_PER * 128 + js[b] * 64
                for o_, b_ in zip(outs, bufs[b]):
                    pltpu.make_async_copy(
                        b_, o_.at[pl.ds(base, 64)], wsems[b]).wait()
        return carry

    lax.fori_loop(0, ROWS_PER // 2, quad, 0)


def _sc_gather(uv, i0, i1, i2, ta, tb, tc):
    dts = [jnp.int32 if v == uv else F32 for v in range(3)]
    fn = pl.kernel(
        functools.partial(_gather_body, uv),
        out_type=[jax.ShapeDtypeStruct((E_PAD, H), dt) for dt in dts],
        mesh=_MESH,
        scratch_types=[pltpu.VMEM((ROWS_PER, 128), jnp.int32)] * 3 +
                      [pltpu.VMEM((64, H), dt) for dt in dts] * 4 +
                      [pltpu.SemaphoreType.DMA] * 8,
        compiler_params=_SC_PARAMS,
    )
    return fn(i0, i1, i2, ta, tb, tc)


def _sums_body(ex2, hw2, ix2, out, exb, hwb, ixb, s1, s2):
    w = _wid()
    zero16 = jnp.zeros((16,), F32)

    def z(i, carry):
        s1[pl.ds(i * 16, 16)] = zero16
        s2[pl.ds(i * 16, 16)] = zero16
        return carry

    lax.fori_loop(0, N // 16, z, 0)
    _stage_rows(ex2, exb, w)
    _stage_rows(hw2, hwb, w)
    _stage_rows(ix2, ixb, w)

    def row(j, carry):
        for k in range(8):
            sl = pl.ds(k * 16, 16)
            i16 = ixb[j, sl]
            e16 = exb[j, sl]
            h16 = hwb[j, sl]
            plsc.addupdate_scatter(s1, [i16], e16)
            plsc.addupdate_scatter(s2, [i16], e16 * h16)
        return carry

    lax.fori_loop(0, _nrows(w), row, 0)
    pltpu.sync_copy(s1, out.at[pl.ds(w * N, N)])
    pltpu.sync_copy(s2, out.at[pl.ds((NW + w) * N, N)])


def _sc_sums(ex2, hw2, ix2):
    fn = pl.kernel(
        _sums_body,
        out_type=jax.ShapeDtypeStruct((2 * NW * N,), F32),
        mesh=_MESH,
        scratch_types=[pltpu.VMEM((ROWS_PER, 128), F32),
                       pltpu.VMEM((ROWS_PER, 128), F32),
                       pltpu.VMEM((ROWS_PER, 128), jnp.int32),
                       pltpu.VMEM((N,), F32),
                       pltpu.VMEM((N,), F32)],
        compiler_params=_SC_PARAMS,
    )
    return fn(ex2, hw2, ix2)


def _alpha_body(ex2, hw2, ix2, ssum, shw, alpha,
                exb, hwb, ixb, sN, gN, abuf):
    w = _wid()
    pltpu.sync_copy(ssum, sN)
    pltpu.sync_copy(shw, gN)
    _stage_rows(ex2, exb, w)
    _stage_rows(hw2, hwb, w)
    _stage_rows(ix2, ixb, w)

    def row(j, carry):
        for k in range(8):
            sl = pl.ds(k * 16, 16)
            i16 = ixb[j, sl]
            e16 = exb[j, sl]
            h16 = hwb[j, sl]
            sg = plsc.load_gather(sN, [i16])
            gg = plsc.load_gather(gN, [i16])
            abuf[sl] = e16 * h16 / (gg + 1e-5 * sg + 1e-21)
        base = (w * ROWS_PER + j) * 128
        pltpu.sync_copy(abuf, alpha.at[pl.ds(base, 128)])
        return carry

    lax.fori_loop(0, _nrows(w), row, 0)


def _sc_alpha(ex2, hw2, ix2, ssum, shw):
    fn = pl.kernel(
        _alpha_body,
        out_type=jax.ShapeDtypeStruct((E_PAD,), F32),
        mesh=_MESH,
        scratch_types=[pltpu.VMEM((ROWS_PER, 128), F32),
                       pltpu.VMEM((ROWS_PER, 128), F32),
                       pltpu.VMEM((ROWS_PER, 128), jnp.int32),
                       pltpu.VMEM((N,), F32),
                       pltpu.VMEM((N,), F32),
                       pltpu.VMEM((128,), F32)],
        compiler_params=_SC_PARAMS,
    )
    return fn(ex2, hw2, ix2, ssum, shw)


def _scatter_body(m, ix2, acc, ixb, ixt0, ixt1, mb0, mb1,
                  semr0, semr1, sems0, sems1, spacc):
    c = lax.axis_index("c")
    s = lax.axis_index("s")

    pltpu.sync_copy(ix2.at[pl.ds(s * SROWS, SROWS)], ixb)
    nrows = jnp.clip(R - s * SROWS, 0, SROWS).astype(jnp.int32)
    lo = c * NHALF
    mbs = (mb0, mb1)
    ixts = (ixt0, ixt1)
    rsems = (semr0, semr1)
    ssems = (sems0, sems1)
    zero16 = jnp.zeros((16,), F32)

    def zm(i, carry):
        for k in range(8):
            mb0[i, pl.ds(k * 16, 16)] = zero16
        return carry

    lax.fori_loop(0, 128, zm, 0)
    pltpu.sync_copy(mb0, spacc.at[pl.ds(s * 328, 128)])
    pltpu.sync_copy(mb0, spacc.at[pl.ds(s * 328 + 128, 128)])
    pltpu.sync_copy(mb0.at[pl.ds(0, 72)],
                    spacc.at[pl.ds(s * 328 + 256, 72)])
    plsc.subcore_barrier()

    def pair(j2, carry):
        js = (2 * j2, 2 * j2 + 1)
        for b in range(2):
            @pl.when(js[b] < nrows)
            def _(b=b):
                base = (s * SROWS + js[b]) * 128
                pltpu.async_copy(m.at[pl.ds(base, 128)], mbs[b], rsems[b])
        for b in range(2):
            @pl.when(js[b] < nrows)
            def _(b=b):
                for k in range(8):
                    sl = pl.ds(k * 16, 16)
                    i16 = ixb[js[b], sl]
                    iloc = i16 - lo
                    ok = (iloc >= 0) & (iloc < NHALF)
                    ixts[b][sl] = jnp.where(ok, iloc, NHALF + 120)
                base = (s * SROWS + js[b]) * 128
                pltpu.make_async_copy(
                    m.at[pl.ds(base, 128)], mbs[b], rsems[b]).wait()
                pltpu.async_copy(mbs[b], spacc.at[ixts[b]], ssems[b],
                                 add=True)
        for b in range(2):
            @pl.when(js[b] < nrows)
            def _(b=b):
                pltpu.make_async_copy(mbs[b], spacc.at[ixts[b]],
                                      ssems[b]).wait()
        return carry

    lax.fori_loop(0, SROWS // 2, pair, 0)
    plsc.subcore_barrier()
    for q in range(2):
        start = s * 328 + q * 128
        pltpu.sync_copy(spacc.at[pl.ds(start, 128)], mb0)
        pltpu.sync_copy(mb0, acc.at[c, pl.ds(start, 128)])
    start = s * 328 + 256
    pltpu.sync_copy(spacc.at[pl.ds(start, 72)], mb0.at[pl.ds(0, 72)])
    pltpu.sync_copy(mb0.at[pl.ds(0, 72)], acc.at[c, pl.ds(start, 72)])


def _sc_scatter(m, ix2):
    fn = pl.kernel(
        _scatter_body,
        out_type=jax.ShapeDtypeStruct((2, ACC_ROWS, H), F32),
        mesh=_MESH,
        scratch_types=[pltpu.VMEM((SROWS, 128), jnp.int32),
                       pltpu.VMEM((128,), jnp.int32),
                       pltpu.VMEM((128,), jnp.int32),
                       pltpu.VMEM((128, H), F32),
                       pltpu.VMEM((128, H), F32),
                       pltpu.SemaphoreType.DMA,
                       pltpu.SemaphoreType.DMA,
                       pltpu.SemaphoreType.DMA,
                       pltpu.SemaphoreType.DMA,
                       pltpu.VMEM_SHARED((ACC_ROWS, H), F32)],
        compiler_params=_SC_PARAMS,
    )
    return fn(m, ix2)


def _tri_stage(x_tables, hi2d, ha, hw2, uv, p, pre):
    t, ta, tb, tc = x_tables
    i0, i1, i2 = hi2d
    g0, g1, g2 = _sc_gather(uv, i0, i1, i2, ta, tb, tc)
    m, ex = _tc_msg(uv, g0, g1, g2, ha, p, pre)
    ex2 = ex.reshape(R_PAD, 128)
    ix2 = (i0, i1, i2)[uv]
    parts = _sc_sums(ex2, hw2, ix2)
    sp = parts[:NW * N].reshape(NW, N)
    gp = parts[NW * N:].reshape(NW, N)
    ssum, shw = _tc_reduce(sp, gp)
    alpha = _sc_alpha(ex2, hw2, ix2, ssum, shw)
    msc = _tc_scale(m, alpha.reshape(E_PAD, 1))
    acc = _sc_scatter(msc, ix2)
    return t, acc


def kernel(x, hyperedge_index, hyperedge_attr, hyperedge_weight, params):
    p = params
    hi = hyperedge_index.astype(jnp.int32)
    pad2d = lambda a: jnp.pad(a.reshape(R, 128), ((0, R_PAD - R), (0, 0)))
    hi2d = tuple(pad2d(hi[v]) for v in range(3))
    hw2 = pad2d(hyperedge_weight)

    t0, xi, xj, tkr = _tc_prep0(x, p)
    t0, acc1 = _tri_stage((t0, xi, xj, tkr), hi2d, hyperedge_attr,
                          hw2, 2, p, "outp")
    x1, tir, yj, yk = _tc_update1(t0, acc1, p)
    x1, acc2 = _tri_stage((x1, tir, yj, yk), hi2d, hyperedge_attr,
                          hw2, 0, p, "inp")
    return _tc_update2(x1, acc2, p)
